# baseline (device time: 85702 ns/iter reference)
import jax
import jax.numpy as jnp
from jax import lax
from jax.experimental import pallas as pl
from jax.experimental.pallas import tpu as pltpu

N_DEV = 4
M = 1024
D = 1024


def kernel(partial, gamma):
    x = partial[0]
    g = gamma.reshape(1, D)

    def body(x_ref, g_ref, out_ref, send_buf, recv_buf, send_sems, recv_sems):
        my = lax.axis_index("i")
        left = (my + N_DEV - 1) % N_DEV
        right = (my + 1) % N_DEV

        barrier_sem = pltpu.get_barrier_semaphore()
        for nbr in (left, right):
            pl.semaphore_signal(
                barrier_sem, inc=1,
                device_id=(nbr,), device_id_type=pl.DeviceIdType.MESH,
            )
        pl.semaphore_wait(barrier_sem, 2)

        for h in range(N_DEV - 1):
            c = (my + N_DEV - 1 - h) % N_DEV
            local = x_ref[pl.ds(c * M, M), :].astype(jnp.bfloat16)
            if h == 0:
                send_buf[h, :, :] = local
            else:
                send_buf[h, :, :] = recv_buf[h - 1, :, :] + local
            rdma = pltpu.make_async_remote_copy(
                src_ref=send_buf.at[h],
                dst_ref=recv_buf.at[h],
                send_sem=send_sems.at[h],
                recv_sem=recv_sems.at[h],
                device_id=(right,),
                device_id_type=pl.DeviceIdType.MESH,
            )
            rdma.start()
            rdma.wait()

        y = (
            recv_buf[N_DEV - 2, :, :]
            + x_ref[pl.ds(my * M, M), :].astype(jnp.bfloat16)
        ).astype(jnp.float32)
        ms = jnp.mean(y * y, axis=-1, keepdims=True)
        out_ref[:, :] = y * lax.rsqrt(ms + 1e-6) * g_ref[:, :]

    return pl.pallas_call(
        body,
        out_shape=jax.ShapeDtypeStruct((M, D), jnp.float32),
        in_specs=[
            pl.BlockSpec(memory_space=pltpu.VMEM),
            pl.BlockSpec(memory_space=pltpu.VMEM),
        ],
        out_specs=pl.BlockSpec(memory_space=pltpu.VMEM),
        scratch_shapes=[
            pltpu.VMEM((N_DEV - 1, M, D), jnp.bfloat16),
            pltpu.VMEM((N_DEV - 1, M, D), jnp.bfloat16),
            pltpu.SemaphoreType.DMA((N_DEV - 1,)),
            pltpu.SemaphoreType.DMA((N_DEV - 1,)),
        ],
        compiler_params=pltpu.CompilerParams(collective_id=0),
    )(x, g)


# device time: 52138 ns/iter; 1.6438x vs baseline; 1.6438x over previous
import jax
import jax.numpy as jnp
from jax import lax
from jax.experimental import pallas as pl
from jax.experimental.pallas import tpu as pltpu

N_DEV = 4
M = 1024
H = M // 2
D = 1024


def kernel(partial, gamma):
    x = partial[0]
    g = gamma.reshape(1, D)

    def body(x_ref, g_ref, out_ref,
             send_r, recv_r, send_l, recv_l,
             ssem_r, rsem_r, ssem_l, rsem_l):
        my = lax.axis_index("i")
        left = (my + N_DEV - 1) % N_DEV
        right = (my + 1) % N_DEV

        barrier_sem = pltpu.get_barrier_semaphore()
        for nbr in (left, right):
            pl.semaphore_signal(
                barrier_sem, inc=1,
                device_id=(nbr,), device_id_type=pl.DeviceIdType.MESH,
            )
        pl.semaphore_wait(barrier_sem, 2)

        def mk(direction, h):
            buf_s, buf_r, sem_s, sem_r, tgt = (
                (send_r, recv_r, ssem_r, rsem_r, right)
                if direction == 0
                else (send_l, recv_l, ssem_l, rsem_l, left)
            )
            return pltpu.make_async_remote_copy(
                src_ref=buf_s.at[h],
                dst_ref=buf_r.at[h],
                send_sem=sem_s.at[h],
                recv_sem=sem_r.at[h],
                device_id=(tgt,),
                device_id_type=pl.DeviceIdType.MESH,
            )

        for h in range(N_DEV - 1):
            cr = (my + N_DEV - 1 - h) % N_DEV
            cl = (my + 1 + h) % N_DEV
            loc_r = x_ref[pl.ds(cr * M, H), :].astype(jnp.bfloat16)
            loc_l = x_ref[pl.ds(cl * M + H, H), :].astype(jnp.bfloat16)
            if h == 0:
                send_r[h, :, :] = loc_r
                send_l[h, :, :] = loc_l
            else:
                send_r[h, :, :] = recv_r[h - 1, :, :] + loc_r
                send_l[h, :, :] = recv_l[h - 1, :, :] + loc_l
            rdma_r = mk(0, h)
            rdma_l = mk(1, h)
            rdma_r.start()
            rdma_l.start()
            if h < N_DEV - 2:
                rdma_r.wait_recv()
                rdma_l.wait_recv()

        mk(0, N_DEV - 2).wait_recv()
        yr = (
            recv_r[N_DEV - 2, :, :]
            + x_ref[pl.ds(my * M, H), :].astype(jnp.bfloat16)
        ).astype(jnp.float32)
        ms_r = jnp.mean(yr * yr, axis=-1, keepdims=True)
        out_ref[0:H, :] = yr * lax.rsqrt(ms_r + 1e-6) * g_ref[:, :]

        mk(1, N_DEV - 2).wait_recv()
        yl = (
            recv_l[N_DEV - 2, :, :]
            + x_ref[pl.ds(my * M + H, H), :].astype(jnp.bfloat16)
        ).astype(jnp.float32)
        ms_l = jnp.mean(yl * yl, axis=-1, keepdims=True)
        out_ref[H:M, :] = yl * lax.rsqrt(ms_l + 1e-6) * g_ref[:, :]

        for h in range(N_DEV - 1):
            mk(0, h).wait_send()
            mk(1, h).wait_send()

    return pl.pallas_call(
        body,
        out_shape=jax.ShapeDtypeStruct((M, D), jnp.float32),
        in_specs=[
            pl.BlockSpec(memory_space=pltpu.VMEM),
            pl.BlockSpec(memory_space=pltpu.VMEM),
        ],
        out_specs=pl.BlockSpec(memory_space=pltpu.VMEM),
        scratch_shapes=[
            pltpu.VMEM((N_DEV - 1, H, D), jnp.bfloat16),
            pltpu.VMEM((N_DEV - 1, H, D), jnp.bfloat16),
            pltpu.VMEM((N_DEV - 1, H, D), jnp.bfloat16),
            pltpu.VMEM((N_DEV - 1, H, D), jnp.bfloat16),
            pltpu.SemaphoreType.DMA((N_DEV - 1,)),
            pltpu.SemaphoreType.DMA((N_DEV - 1,)),
            pltpu.SemaphoreType.DMA((N_DEV - 1,)),
            pltpu.SemaphoreType.DMA((N_DEV - 1,)),
        ],
        compiler_params=pltpu.CompilerParams(collective_id=0),
    )(x, g)


# device time: 47724 ns/iter; 1.7958x vs baseline; 1.0925x over previous
import jax
import jax.numpy as jnp
from jax import lax
from jax.experimental import pallas as pl
from jax.experimental.pallas import tpu as pltpu

N_DEV = 4
M = 1024
H = M // 2
NSUB = 2
Q = H // NSUB
D = 1024
NHOP = N_DEV - 1


def kernel(partial, gamma):
    x = partial[0]
    g = gamma.reshape(1, D)

    def body(x_ref, g_ref, out_ref,
             send_r, recv_r, send_l, recv_l,
             ssem_r, rsem_r, ssem_l, rsem_l):
        my = lax.axis_index("i")
        left = (my + N_DEV - 1) % N_DEV
        right = (my + 1) % N_DEV

        barrier_sem = pltpu.get_barrier_semaphore()
        for nbr in (left, right):
            pl.semaphore_signal(
                barrier_sem, inc=1,
                device_id=(nbr,), device_id_type=pl.DeviceIdType.MESH,
            )
        pl.semaphore_wait(barrier_sem, 2)

        def mk(direction, h, sub):
            buf_s, buf_r, sem_s, sem_r, tgt = (
                (send_r, recv_r, ssem_r, rsem_r, right)
                if direction == 0
                else (send_l, recv_l, ssem_l, rsem_l, left)
            )
            return pltpu.make_async_remote_copy(
                src_ref=buf_s.at[h, sub],
                dst_ref=buf_r.at[h, sub],
                send_sem=sem_s.at[h, sub],
                recv_sem=sem_r.at[h, sub],
                device_id=(tgt,),
                device_id_type=pl.DeviceIdType.MESH,
            )

        def loc(direction, c, sub):
            base = c * M + direction * H + sub * Q
            return x_ref[pl.ds(base, Q), :].astype(jnp.bfloat16)

        for h in range(NHOP):
            cr = (my + N_DEV - 1 - h) % N_DEV
            cl = (my + 1 + h) % N_DEV
            for sub in range(NSUB):
                if h == 0:
                    send_r[h, sub, :, :] = loc(0, cr, sub)
                    send_l[h, sub, :, :] = loc(1, cl, sub)
                else:
                    mk(0, h - 1, sub).wait_recv()
                    send_r[h, sub, :, :] = recv_r[h - 1, sub, :, :] + loc(0, cr, sub)
                    mk(1, h - 1, sub).wait_recv()
                    send_l[h, sub, :, :] = recv_l[h - 1, sub, :, :] + loc(1, cl, sub)
                mk(0, h, sub).start()
                mk(1, h, sub).start()

        for sub in range(NSUB):
            mk(0, NHOP - 1, sub).wait_recv()
            yr = (recv_r[NHOP - 1, sub, :, :] + loc(0, my, sub)).astype(jnp.float32)
            ms = jnp.mean(yr * yr, axis=-1, keepdims=True)
            out_ref[pl.ds(sub * Q, Q), :] = yr * lax.rsqrt(ms + 1e-6) * g_ref[:, :]

            mk(1, NHOP - 1, sub).wait_recv()
            yl = (recv_l[NHOP - 1, sub, :, :] + loc(1, my, sub)).astype(jnp.float32)
            ms = jnp.mean(yl * yl, axis=-1, keepdims=True)
            out_ref[pl.ds(H + sub * Q, Q), :] = yl * lax.rsqrt(ms + 1e-6) * g_ref[:, :]

        for h in range(NHOP):
            for sub in range(NSUB):
                mk(0, h, sub).wait_send()
                mk(1, h, sub).wait_send()

    return pl.pallas_call(
        body,
        out_shape=jax.ShapeDtypeStruct((M, D), jnp.float32),
        in_specs=[
            pl.BlockSpec(memory_space=pltpu.VMEM),
            pl.BlockSpec(memory_space=pltpu.VMEM),
        ],
        out_specs=pl.BlockSpec(memory_space=pltpu.VMEM),
        scratch_shapes=[
            pltpu.VMEM((NHOP, NSUB, Q, D), jnp.bfloat16),
            pltpu.VMEM((NHOP, NSUB, Q, D), jnp.bfloat16),
            pltpu.VMEM((NHOP, NSUB, Q, D), jnp.bfloat16),
            pltpu.VMEM((NHOP, NSUB, Q, D), jnp.bfloat16),
            pltpu.SemaphoreType.DMA((NHOP, NSUB)),
            pltpu.SemaphoreType.DMA((NHOP, NSUB)),
            pltpu.SemaphoreType.DMA((NHOP, NSUB)),
            pltpu.SemaphoreType.DMA((NHOP, NSUB)),
        ],
        compiler_params=pltpu.CompilerParams(collective_id=0),
    )(x, g)
